# baseline (speedup 1.0000x reference)
import jax
import jax.numpy as jnp
from jax.experimental import pallas as pl

_B = 4096
_D = 128
_S = 16384
_TEMP = 0.05
_INV_TEMP = 1.0 / _TEMP
_SHIFT = _INV_TEMP

_BB = 512
_SC = 2048
_NB = _B // _BB
_NS = _S // _SC


def _ce_kernel(x_ref, t_ref, f_ref, out_ref):
    i = pl.program_id(0)
    x = x_ref[...]
    nrm = jnp.sqrt(jnp.sum(x * x, axis=1, keepdims=True))
    xn = x / jnp.maximum(nrm, 1e-12)
    tgt_idx = t_ref[0]

    xnb = xn.astype(jnp.bfloat16)

    def body(j, carry):
        acc, tlogit = carry
        f = f_ref[pl.ds(j * _SC, _SC), :]
        l = jax.lax.dot_general(
            xnb, f, (((1,), (1,)), ((), ())),
            preferred_element_type=jnp.float32) * _INV_TEMP
        acc = acc + jnp.sum(jnp.exp(l - _SHIFT), axis=1, keepdims=True)
        col = jax.lax.broadcasted_iota(jnp.int32, (_BB, _SC), 1) + j * _SC
        hit = col == tgt_idx
        tlogit = tlogit + jnp.sum(jnp.where(hit, l, 0.0), axis=1, keepdims=True)
        return acc, tlogit

    acc0 = jnp.zeros((_BB, 1), jnp.float32)
    acc, tlogit = jax.lax.fori_loop(0, _NS, body, (acc0, acc0))
    partial = jnp.sum(_SHIFT + jnp.log(acc) - tlogit).reshape(1, 1)

    @pl.when(i == 0)
    def _():
        out_ref[...] = partial

    @pl.when(i > 0)
    def _():
        out_ref[...] = out_ref[...] + partial


def kernel(inputs, k_inputs, targets, features):
    del k_inputs
    t3 = targets.astype(jnp.int32).reshape(_NB, _BB, 1)
    features = features.astype(jnp.bfloat16)
    out = pl.pallas_call(
        _ce_kernel,
        grid=(_NB,),
        in_specs=[
            pl.BlockSpec((_BB, _D), lambda i: (i, 0)),
            pl.BlockSpec((1, _BB, 1), lambda i: (i, 0, 0)),
            pl.BlockSpec((_S, _D), lambda i: (0, 0)),
        ],
        out_specs=pl.BlockSpec((1, 1), lambda i: (0, 0)),
        out_shape=jax.ShapeDtypeStruct((1, 1), jnp.float32),
    )(inputs, t3, features)
    return out[0, 0] / _B


# speedup vs baseline: 1.2576x; 1.2576x over previous
import functools

import jax
import jax.numpy as jnp
from jax import lax
from jax.experimental import pallas as pl
from jax.experimental.pallas import tpu as pltpu
from jax.experimental.pallas import tpu_sc as plsc

_B = 4096
_D = 128
_S = 16384
_TEMP = 0.05
_INV_TEMP = 1.0 / _TEMP
_LOG2E = 1.4426950408889634
_SHIFT2 = _INV_TEMP * _LOG2E

_BB = 512
_SC = 2048
_NB = _B // _BB
_NS = _S // _SC

_NC = 2
_NSUB = 16
_NW = _NC * _NSUB
_BPW = _B // _NW


def _logz_kernel(x_ref, f_ref, out_ref):
    i = pl.program_id(0)
    x = x_ref[...]
    nrm = jnp.sqrt(jnp.sum(x * x, axis=1, keepdims=True))
    xs = (x / jnp.maximum(nrm, 1e-12) * (_INV_TEMP * _LOG2E)).astype(jnp.bfloat16)

    def body(j, acc):
        f = f_ref[pl.ds(j * _SC, _SC), :]
        l2 = jax.lax.dot_general(
            xs, f, (((1,), (1,)), ((), ())),
            preferred_element_type=jnp.float32)
        return acc + jnp.sum(jnp.exp2(l2 - _SHIFT2), axis=1, keepdims=True)

    acc = jax.lax.fori_loop(0, _NS, body, jnp.zeros((_BB, 1), jnp.float32))
    partial = jnp.sum(_INV_TEMP + jnp.log(acc)).reshape(1, 1)

    @pl.when(i == 0)
    def _():
        out_ref[...] = partial

    @pl.when(i > 0)
    def _():
        out_ref[...] = out_ref[...] + partial


def _tc_logz_sum(x, features):
    out = pl.pallas_call(
        _logz_kernel,
        grid=(_NB,),
        in_specs=[
            pl.BlockSpec((_BB, _D), lambda i: (i, 0)),
            pl.BlockSpec((_S, _D), lambda i: (0, 0)),
        ],
        out_specs=pl.BlockSpec((1, 1), lambda i: (0, 0)),
        out_shape=jax.ShapeDtypeStruct((1, 1), jnp.float32),
    )(x, features)
    return out[0, 0]


def _sc_tgt_kernel(x_hbm, t_hbm, f_hbm, out_hbm,
                   idx_v, xr_v, gr_v, acc_v, sem):
    c = lax.axis_index("c")
    s = lax.axis_index("s")
    wid = s * _NC + c
    base = wid * _BPW
    pltpu.sync_copy(t_hbm.at[pl.ds(base, _BPW)], idx_v)
    cp = pltpu.async_copy(f_hbm.at[idx_v], gr_v, sem)
    pltpu.sync_copy(x_hbm.at[pl.ds(base, _BPW)], xr_v)
    cp.wait()

    def _lane_sum(v):
        t = v[0] + v[1]
        for k in range(2, 16):
            t = t + v[k]
        return t

    def row_body(r, tot):
        dot = jnp.zeros((16,), jnp.float32)
        ssq = jnp.zeros((16,), jnp.float32)
        for cc in range(_D // 16):
            xv = xr_v[r, pl.ds(cc * 16, 16)]
            gv = gr_v[r, pl.ds(cc * 16, 16)]
            dot = dot + xv * gv
            ssq = ssq + xv * xv
        sd = _lane_sum(dot)
        sq = jnp.maximum(_lane_sum(ssq), 1e-24)
        bits = lax.bitcast_convert_type(sq, jnp.int32)
        bits = jnp.int32(0x5F3759DF) - lax.shift_right_logical(bits, 1)
        y = lax.bitcast_convert_type(bits, jnp.float32)
        for _ in range(3):
            y = y * (1.5 - 0.5 * sq * y * y)
        return tot + sd * y

    total = lax.fori_loop(0, _BPW, row_body, jnp.float32(0.0))
    acc_v[...] = jnp.full((16,), 0.0625, jnp.float32) * total
    pltpu.sync_copy(acc_v, out_hbm.at[wid])


def _sc_tgt_sums(x, targets, features):
    mesh = plsc.VectorSubcoreMesh(core_axis_name="c", subcore_axis_name="s")
    run = functools.partial(
        pl.kernel,
        mesh=mesh,
        out_type=jax.ShapeDtypeStruct((_NW, 16), jnp.float32),
        scratch_types=[
            pltpu.VMEM((_BPW,), jnp.int32),
            pltpu.VMEM((_BPW, _D), jnp.float32),
            pltpu.VMEM((_BPW, _D), jnp.float32),
            pltpu.VMEM((16,), jnp.float32),
            pltpu.SemaphoreType.DMA,
        ],
    )(_sc_tgt_kernel)
    return run(x, targets, features)


def kernel(inputs, k_inputs, targets, features):
    del k_inputs
    t32 = targets.astype(jnp.int32)
    fb = features.astype(jnp.bfloat16)
    tgt_parts = _sc_tgt_sums(inputs, t32, features)
    logz_sum = _tc_logz_sum(inputs, fb)
    tgt_sum = jnp.sum(tgt_parts)
    return (logz_sum - _INV_TEMP * tgt_sum) / _B


# speedup vs baseline: 1.7080x; 1.3581x over previous
import functools

import jax
import jax.numpy as jnp
from jax import lax
from jax.experimental import pallas as pl
from jax.experimental.pallas import tpu as pltpu
from jax.experimental.pallas import tpu_sc as plsc

_B = 4096
_D = 128
_S = 16384
_TEMP = 0.05
_INV_TEMP = 1.0 / _TEMP
_LOG2E = 1.4426950408889634
_SHIFT2 = _INV_TEMP * _LOG2E

_BB = 512
_SC = 2048
_NB = _B // _BB
_NS = _S // _SC

_NC = 2
_NSUB = 16
_NW = _NC * _NSUB
_BPW = _B // _NW


def _logz_kernel(x_ref, f_ref, out_ref):
    i = pl.program_id(0)
    x = x_ref[...]
    nrm = jnp.sqrt(jnp.sum(x * x, axis=1, keepdims=True))
    xs = (x / jnp.maximum(nrm, 1e-12) * (_INV_TEMP * _LOG2E)).astype(jnp.bfloat16)

    acc = jnp.zeros((_BB, 1), jnp.float32)
    for j in range(_NS):
        f = f_ref[pl.ds(j * _SC, _SC), :]
        l2 = jax.lax.dot_general(
            xs, f, (((1,), (1,)), ((), ())),
            preferred_element_type=jnp.float32)
        acc = acc + jnp.sum(jnp.exp2(l2), axis=1, keepdims=True)

    partial = jnp.sum(jnp.log(acc)).reshape(1, 1)

    @pl.when(i == 0)
    def _():
        out_ref[...] = partial

    @pl.when(i > 0)
    def _():
        out_ref[...] = out_ref[...] + partial


def _tc_logz_sum(x, features):
    out = pl.pallas_call(
        _logz_kernel,
        grid=(_NB,),
        in_specs=[
            pl.BlockSpec((_BB, _D), lambda i: (i, 0)),
            pl.BlockSpec((_S, _D), lambda i: (0, 0)),
        ],
        out_specs=pl.BlockSpec((1, 1), lambda i: (0, 0)),
        out_shape=jax.ShapeDtypeStruct((1, 1), jnp.float32),
    )(x, features)
    return out[0, 0]


def _sc_tgt_kernel(x_hbm, t_hbm, f_hbm, out_hbm,
                   idx_v, xr_v, gr_v, acc_v, sem):
    c = lax.axis_index("c")
    s = lax.axis_index("s")
    wid = s * _NC + c
    base = wid * _BPW
    pltpu.sync_copy(t_hbm.at[pl.ds(base, _BPW)], idx_v)
    cp = pltpu.async_copy(f_hbm.at[idx_v], gr_v, sem)
    pltpu.sync_copy(x_hbm.at[pl.ds(base, _BPW)], xr_v)
    cp.wait()

    def _lane_sum(v):
        t = v[0] + v[1]
        for k in range(2, 16):
            t = t + v[k]
        return t

    def row_body(r, tot):
        dot = jnp.zeros((16,), jnp.float32)
        ssq = jnp.zeros((16,), jnp.float32)
        for cc in range(_D // 16):
            xv = xr_v[r, pl.ds(cc * 16, 16)]
            gv = gr_v[r, pl.ds(cc * 16, 16)]
            dot = dot + xv * gv
            ssq = ssq + xv * xv
        sd = _lane_sum(dot)
        sq = jnp.maximum(_lane_sum(ssq), 1e-24)
        bits = lax.bitcast_convert_type(sq, jnp.int32)
        bits = jnp.int32(0x5F3759DF) - lax.shift_right_logical(bits, 1)
        y = lax.bitcast_convert_type(bits, jnp.float32)
        for _ in range(3):
            y = y * (1.5 - 0.5 * sq * y * y)
        return tot + sd * y

    total = lax.fori_loop(0, _BPW, row_body, jnp.float32(0.0))
    acc_v[...] = jnp.full((16,), 0.0625, jnp.float32) * total
    pltpu.sync_copy(acc_v, out_hbm.at[wid])


def _sc_tgt_sums(x, targets, features):
    mesh = plsc.VectorSubcoreMesh(core_axis_name="c", subcore_axis_name="s")
    run = functools.partial(
        pl.kernel,
        mesh=mesh,
        out_type=jax.ShapeDtypeStruct((_NW, 16), jnp.float32),
        scratch_types=[
            pltpu.VMEM((_BPW,), jnp.int32),
            pltpu.VMEM((_BPW, _D), jnp.float32),
            pltpu.VMEM((_BPW, _D), jnp.float32),
            pltpu.VMEM((16,), jnp.float32),
            pltpu.SemaphoreType.DMA,
        ],
    )(_sc_tgt_kernel)
    return run(x, targets, features)


def kernel(inputs, k_inputs, targets, features):
    del k_inputs
    t32 = targets.astype(jnp.int32)
    fb = features.astype(jnp.bfloat16)
    tgt_parts = _sc_tgt_sums(inputs, t32, features)
    logz_sum = _tc_logz_sum(inputs, fb)
    tgt_sum = jnp.sum(tgt_parts)
    return (logz_sum - _INV_TEMP * tgt_sum) / _B


# speedup vs baseline: 1.8748x; 1.0977x over previous
import functools

import jax
import jax.numpy as jnp
from jax import lax
from jax.experimental import pallas as pl
from jax.experimental.pallas import tpu as pltpu
from jax.experimental.pallas import tpu_sc as plsc

_B = 4096
_D = 128
_S = 16384
_TEMP = 0.05
_INV_TEMP = 1.0 / _TEMP
_LOG2E = 1.4426950408889634
_SHIFT2 = _INV_TEMP * _LOG2E

_BB = 1024
_SC = 2048
_NB = _B // _BB
_NS = _S // _SC

_NC = 2
_NSUB = 16
_NW = _NC * _NSUB
_BPW = _B // _NW


def _logz_kernel(x_ref, f_ref, out_ref):
    i = pl.program_id(0)
    x = x_ref[...]
    nrm = jnp.sqrt(jnp.sum(x * x, axis=1, keepdims=True))
    xs = (x / jnp.maximum(nrm, 1e-12) * (_INV_TEMP * _LOG2E)).astype(jnp.bfloat16)

    acc = jnp.zeros((_BB, 1), jnp.float32)
    for j in range(_NS):
        f = f_ref[pl.ds(j * _SC, _SC), :].astype(jnp.bfloat16)
        l2 = jax.lax.dot_general(
            xs, f, (((1,), (1,)), ((), ())),
            preferred_element_type=jnp.float32)
        acc = acc + jnp.sum(jnp.exp2(l2), axis=1, keepdims=True)

    partial = jnp.sum(jnp.log(acc)).reshape(1, 1)

    @pl.when(i == 0)
    def _():
        out_ref[...] = partial

    @pl.when(i > 0)
    def _():
        out_ref[...] = out_ref[...] + partial


def _tc_logz_sum(x, features):
    out = pl.pallas_call(
        _logz_kernel,
        grid=(_NB,),
        in_specs=[
            pl.BlockSpec((_BB, _D), lambda i: (i, 0)),
            pl.BlockSpec((_S, _D), lambda i: (0, 0)),
        ],
        out_specs=pl.BlockSpec((1, 1), lambda i: (0, 0)),
        out_shape=jax.ShapeDtypeStruct((1, 1), jnp.float32),
    )(x, features)
    return out[0, 0]


def _sc_tgt_kernel(x_hbm, t_hbm, f_hbm, out_hbm,
                   idx_v, xr_v, gr_v, acc_v, sem):
    c = lax.axis_index("c")
    s = lax.axis_index("s")
    wid = s * _NC + c
    base = wid * _BPW
    pltpu.sync_copy(t_hbm.at[pl.ds(base, _BPW)], idx_v)
    cp = pltpu.async_copy(f_hbm.at[idx_v], gr_v, sem)
    pltpu.sync_copy(x_hbm.at[pl.ds(base, _BPW)], xr_v)
    cp.wait()

    def _lane_sum(v):
        t = v[0] + v[1]
        for k in range(2, 16):
            t = t + v[k]
        return t

    def row_body(r, tot):
        dot = jnp.zeros((16,), jnp.float32)
        ssq = jnp.zeros((16,), jnp.float32)
        for cc in range(_D // 16):
            xv = xr_v[r, pl.ds(cc * 16, 16)]
            gv = gr_v[r, pl.ds(cc * 16, 16)]
            dot = dot + xv * gv
            ssq = ssq + xv * xv
        sd = _lane_sum(dot)
        sq = jnp.maximum(_lane_sum(ssq), 1e-24)
        bits = lax.bitcast_convert_type(sq, jnp.int32)
        bits = jnp.int32(0x5F3759DF) - lax.shift_right_logical(bits, 1)
        y = lax.bitcast_convert_type(bits, jnp.float32)
        for _ in range(3):
            y = y * (1.5 - 0.5 * sq * y * y)
        return tot + sd * y

    total = lax.fori_loop(0, _BPW, row_body, jnp.float32(0.0))
    acc_v[...] = jnp.full((16,), 0.0625, jnp.float32) * total
    pltpu.sync_copy(acc_v, out_hbm.at[wid])


def _sc_tgt_sums(x, targets, features):
    mesh = plsc.VectorSubcoreMesh(core_axis_name="c", subcore_axis_name="s")
    run = functools.partial(
        pl.kernel,
        mesh=mesh,
        out_type=jax.ShapeDtypeStruct((_NW, 16), jnp.float32),
        scratch_types=[
            pltpu.VMEM((_BPW,), jnp.int32),
            pltpu.VMEM((_BPW, _D), jnp.float32),
            pltpu.VMEM((_BPW, _D), jnp.float32),
            pltpu.VMEM((16,), jnp.float32),
            pltpu.SemaphoreType.DMA,
        ],
    )(_sc_tgt_kernel)
    return run(x, targets, features)


def kernel(inputs, k_inputs, targets, features):
    del k_inputs
    t32 = targets.astype(jnp.int32)
    tgt_parts = _sc_tgt_sums(inputs, t32, features)
    logz_sum = _tc_logz_sum(inputs, features)
    tgt_sum = jnp.sum(tgt_parts)
    return (logz_sum - _INV_TEMP * tgt_sum) / _B
